# baseline (speedup 1.0000x reference)
import functools
import math

import jax
import jax.numpy as jnp
from jax import lax
from jax.experimental import pallas as pl
from jax.experimental.pallas import tpu as pltpu
from jax.experimental.pallas import tpu_sc as plsc

D_MODEL = 64
SCALE = math.sqrt(D_MODEL)
NUM_CORES = 2
NUM_SUBCORES = 16
NUM_WORKERS = NUM_CORES * NUM_SUBCORES
LANES = 16
ROWS_PER_CHUNK = 4


def _emb_call(S, T):
    rows_per_w = S // NUM_WORKERS
    CH = ROWS_PER_CHUNK * T
    n_chunks = rows_per_w // ROWS_PER_CHUNK
    assert n_chunks % 2 == 0
    mesh = plsc.VectorSubcoreMesh(core_axis_name="c", subcore_axis_name="s")

    @functools.partial(
        pl.kernel,
        mesh=mesh,
        out_type=jax.ShapeDtypeStruct((S, T, D_MODEL), jnp.float32),
        scratch_types=[
            pltpu.VMEM((CH,), jnp.int32),
            pltpu.VMEM((CH,), jnp.int32),
            pltpu.VMEM((2, CH, D_MODEL), jnp.float32),
            pltpu.SemaphoreType.DMA,
            pltpu.SemaphoreType.DMA,
            pltpu.SemaphoreType.DMA,
            pltpu.SemaphoreType.DMA,
            pltpu.SemaphoreType.DMA,
            pltpu.SemaphoreType.DMA,
        ],
        compiler_params=pltpu.CompilerParams(use_tc_tiling_on_sc=False),
    )
    def emb_kernel(x_hbm, table_hbm, out_hbm, idx0, idx1, rows_v,
                   si0, si1, sg0, sg1, so0, so1):
        wid = lax.axis_index("s") * NUM_CORES + lax.axis_index("c")
        base_row = wid * rows_per_w
        idxs = (idx0, idx1)
        isems = (si0, si1)
        gsems = (sg0, sg1)
        osems = (so0, so1)

        def i_descs(ci, b):
            r0 = base_row + ci * ROWS_PER_CHUNK
            return [
                pltpu.make_async_copy(
                    x_hbm.at[r0 + j], idxs[b].at[pl.ds(T * j, T)], isems[b]
                )
                for j in range(ROWS_PER_CHUNK)
            ]

        def g_desc(ci, b):
            return pltpu.make_async_copy(
                table_hbm.at[idxs[b]], rows_v.at[b], gsems[b]
            )

        def o_descs(ci, b):
            r0 = base_row + ci * ROWS_PER_CHUNK
            return [
                pltpu.make_async_copy(
                    rows_v.at[b, pl.ds(T * j, T)], out_hbm.at[r0 + j], osems[b]
                )
                for j in range(ROWS_PER_CHUNK)
            ]

        def scale(b):
            def body(i, c):
                for u in range(8):
                    r = i * 8 + u
                    for j in range(D_MODEL // LANES):
                        sl = pl.ds(j * LANES, LANES)
                        rows_v[b, r, sl] = rows_v[b, r, sl] * SCALE
                return c

            lax.fori_loop(0, CH // 8, body, 0)

        for d in i_descs(0, 0):
            d.start()
        for d in i_descs(0, 0):
            d.wait()
        g_desc(0, 0).start()
        for d in i_descs(1, 1):
            d.start()

        def outer(g, carry):
            for b in (0, 1):
                ci = 2 * g + b
                nb = 1 - b

                @pl.when(ci >= 1)
                def _():
                    for d in o_descs(ci - 1, nb):
                        d.wait()

                @pl.when(ci + 1 < n_chunks)
                def _():
                    for d in i_descs(ci + 1, nb):
                        d.wait()
                    g_desc(ci + 1, nb).start()

                g_desc(ci, b).wait()

                @pl.when(ci + 2 < n_chunks)
                def _():
                    for d in i_descs(ci + 2, b):
                        d.start()

                scale(b)
                for d in o_descs(ci, b):
                    d.start()
            return carry

        lax.fori_loop(0, n_chunks // 2, outer, 0)
        for d in o_descs(n_chunks - 1, 1):
            d.wait()

    return emb_kernel


def kernel(x, table):
    S, T = x.shape
    return _emb_call(S, T)(x.astype(jnp.int32), table)


# speedup vs baseline: 1.3272x; 1.3272x over previous
import functools
import math

import jax
import jax.numpy as jnp
from jax import lax
from jax.experimental import pallas as pl
from jax.experimental.pallas import tpu as pltpu
from jax.experimental.pallas import tpu_sc as plsc

D_MODEL = 64
SCALE = math.sqrt(D_MODEL)
NUM_CORES = 2
NUM_SUBCORES = 16
NUM_WORKERS = NUM_CORES * NUM_SUBCORES
LANES = 16
ROWS_PER_CHUNK = 4


def _emb_call(S, T):
    rows_per_w = S // NUM_WORKERS
    CH = ROWS_PER_CHUNK * T
    n_chunks = rows_per_w // ROWS_PER_CHUNK
    assert n_chunks % 2 == 0
    mesh = plsc.VectorSubcoreMesh(core_axis_name="c", subcore_axis_name="s")

    @functools.partial(
        pl.kernel,
        mesh=mesh,
        out_type=jax.ShapeDtypeStruct((S, T, 2 * D_MODEL), jnp.float32),
        scratch_types=[
            pltpu.VMEM((CH,), jnp.int32),
            pltpu.VMEM((CH,), jnp.int32),
            pltpu.VMEM((2, CH, D_MODEL), jnp.float32),
            pltpu.SemaphoreType.DMA,
            pltpu.SemaphoreType.DMA,
            pltpu.SemaphoreType.DMA,
            pltpu.SemaphoreType.DMA,
            pltpu.SemaphoreType.DMA,
            pltpu.SemaphoreType.DMA,
        ],
        compiler_params=pltpu.CompilerParams(use_tc_tiling_on_sc=False),
    )
    def emb_kernel(x_hbm, table_hbm, out_hbm, idx0, idx1, rows_v,
                   si0, si1, sg0, sg1, so0, so1):
        wid = lax.axis_index("s") * NUM_CORES + lax.axis_index("c")
        base_row = wid * rows_per_w
        idxs = (idx0, idx1)
        isems = (si0, si1)
        gsems = (sg0, sg1)
        osems = (so0, so1)

        def i_descs(ci, b):
            r0 = base_row + ci * ROWS_PER_CHUNK
            return [
                pltpu.make_async_copy(
                    x_hbm.at[r0 + j], idxs[b].at[pl.ds(T * j, T)], isems[b]
                )
                for j in range(ROWS_PER_CHUNK)
            ]

        def g_desc(ci, b):
            return pltpu.make_async_copy(
                table_hbm.at[idxs[b]], rows_v.at[b], gsems[b]
            )

        def o_descs(ci, b):
            r0 = base_row + ci * ROWS_PER_CHUNK
            return [
                pltpu.make_async_copy(
                    rows_v.at[b, pl.ds(T * j, T)],
                    out_hbm.at[r0 + j, pl.ds(0, T), pl.ds(0, D_MODEL)],
                    osems[b],
                )
                for j in range(ROWS_PER_CHUNK)
            ]

        def scale(b):
            def body(i, c):
                for u in range(8):
                    r = i * 8 + u
                    for j in range(D_MODEL // LANES):
                        sl = pl.ds(j * LANES, LANES)
                        rows_v[b, r, sl] = rows_v[b, r, sl] * SCALE
                return c

            lax.fori_loop(0, CH // 8, body, 0)

        for d in i_descs(0, 0):
            d.start()
        for d in i_descs(0, 0):
            d.wait()
        g_desc(0, 0).start()
        for d in i_descs(1, 1):
            d.start()

        def outer(g, carry):
            for b in (0, 1):
                ci = 2 * g + b
                nb = 1 - b

                @pl.when(ci >= 1)
                def _():
                    for d in o_descs(ci - 1, nb):
                        d.wait()

                @pl.when(ci + 1 < n_chunks)
                def _():
                    for d in i_descs(ci + 1, nb):
                        d.wait()
                    g_desc(ci + 1, nb).start()

                g_desc(ci, b).wait()

                @pl.when(ci + 2 < n_chunks)
                def _():
                    for d in i_descs(ci + 2, b):
                        d.start()

                scale(b)
                for d in o_descs(ci, b):
                    d.start()
            return carry

        lax.fori_loop(0, n_chunks // 2, outer, 0)
        for d in o_descs(n_chunks - 1, 1):
            d.wait()

    return emb_kernel


def kernel(x, table):
    S, T = x.shape
    out128 = _emb_call(S, T)(x.astype(jnp.int32), table)
    return lax.slice(out128, (0, 0, 0), (S, T, D_MODEL))
